# Initial kernel scaffold; baseline (speedup 1.0000x reference)
import jax
import jax.numpy as jnp
from jax.experimental import pallas as pl


def kernel(x, edge_index, batch, W1, b1, W2, b2, W3, b3):
    raise NotImplementedError("write your pallas kernel here")


# speedup vs baseline: 5.1891x; 5.1891x over previous
import functools

import jax
import jax.numpy as jnp
from jax import lax
from jax.experimental import pallas as pl
from jax.experimental.pallas import tpu as pltpu
from jax.experimental.pallas import tpu_sc as plsc

N = 10000
NPAD = 10008
D = 512
G = 64
E = 160000
EPAD = 163840
NB = 80
BE = 128
BN = 1112
GRID = NPAD // BN
NCHUNK = 4

def _make_mesh():
    return plsc.VectorSubcoreMesh(core_axis_name="c", subcore_axis_name="s")


def _deg_body(dst3, ones_big, out, didx, ones_v, shared, sem):
    del sem
    cid = lax.axis_index("c")
    sid = lax.axis_index("s")
    pltpu.sync_copy(dst3.at[sid], didx)
    pltpu.sync_copy(ones_big.at[pl.ds(0, BE)], ones_v)

    @pl.when(sid == 0)
    def _():
        pltpu.sync_copy(ones_big, shared)

    plsc.subcore_barrier()

    def body(t, carry):
        pltpu.sync_copy(ones_v, shared.at[didx.at[t]], add=True)
        return carry

    lax.fori_loop(0, NB, body, 0)
    plsc.subcore_barrier()

    @pl.when((sid == 0) & (cid == 0))
    def _():
        pltpu.sync_copy(shared, out)


def _deg_kernel(dst3, ones_big):
    fn = pl.kernel(
        _deg_body,
        mesh=_make_mesh(),
        out_type=jax.ShapeDtypeStruct((NPAD, 128), jnp.float32),
        scratch_types=[
            pltpu.VMEM((NB, BE), jnp.int32),
            pltpu.VMEM((BE, 128), jnp.float32),
            pltpu.VMEM_SHARED((NPAD, 128), jnp.float32),
            pltpu.SemaphoreType.DMA,
        ],
    )
    return fn(dst3, ones_big)


def _scatter_body(hs4, src3, dst3, out, sidx, didx, rows, shared, sem):
    cid = lax.axis_index("c")
    sid = lax.axis_index("s")
    pltpu.sync_copy(src3.at[sid], sidx)
    pltpu.sync_copy(dst3.at[sid], didx)
    for jj in range(NCHUNK // 2):
        chunk = cid * (NCHUNK // 2) + jj

        @pl.when(sid == 0)
        def _():
            pltpu.sync_copy(hs4.at[chunk], shared)

        plsc.subcore_barrier()

        def body(t, carry):
            pltpu.async_copy(hs4.at[chunk].at[sidx.at[t]], rows, sem).wait()
            pltpu.sync_copy(rows, shared.at[didx.at[t]], add=True)
            return carry

        lax.fori_loop(0, NB, body, 0)
        plsc.subcore_barrier()

        @pl.when(sid == 0)
        def _():
            pltpu.sync_copy(shared, out.at[chunk])

        plsc.subcore_barrier()


def _scatter_kernel(hs4, src3, dst3):
    fn = pl.kernel(
        _scatter_body,
        mesh=_make_mesh(),
        out_type=jax.ShapeDtypeStruct((NCHUNK, NPAD, 128), jnp.float32),
        scratch_types=[
            pltpu.VMEM((NB, BE), jnp.int32),
            pltpu.VMEM((NB, BE), jnp.int32),
            pltpu.VMEM((BE, 128), jnp.float32),
            pltpu.VMEM_SHARED((NPAD, 128), jnp.float32),
            pltpu.SemaphoreType.DMA,
        ],
    )
    return fn(hs4, src3, dst3)


def _mm_body(x_ref, dis_ref, w_ref, out_ref):
    xs = x_ref[...] * dis_ref[...]
    h = jnp.dot(xs, w_ref[...], preferred_element_type=jnp.float32)
    for j in range(NCHUNK):
        out_ref[j] = h[:, j * 128:(j + 1) * 128]


def _mm_scaled(x, dis, w):
    return pl.pallas_call(
        _mm_body,
        grid=(GRID,),
        in_specs=[
            pl.BlockSpec((BN, D), lambda i: (i, 0)),
            pl.BlockSpec((BN, 1), lambda i: (i, 0)),
            pl.BlockSpec((D, D), lambda i: (0, 0)),
        ],
        out_specs=pl.BlockSpec((NCHUNK, BN, 128), lambda i: (0, i, 0)),
        out_shape=jax.ShapeDtypeStruct((NCHUNK, NPAD, 128), jnp.float32),
    )(x, dis, w)


def _ew_body(acc_ref, dis_ref, b_ref, res_ref, out_ref):
    h = jnp.concatenate([acc_ref[j] for j in range(NCHUNK)], axis=-1)
    out_ref[...] = res_ref[...] + jax.nn.relu(h * dis_ref[...] + b_ref[...])


def _epilogue(acc4, dis, b, res):
    return pl.pallas_call(
        _ew_body,
        grid=(GRID,),
        in_specs=[
            pl.BlockSpec((NCHUNK, BN, 128), lambda i: (0, i, 0)),
            pl.BlockSpec((BN, 1), lambda i: (i, 0)),
            pl.BlockSpec((1, D), lambda i: (0, 0)),
            pl.BlockSpec((BN, D), lambda i: (i, 0)),
        ],
        out_specs=pl.BlockSpec((BN, D), lambda i: (i, 0)),
        out_shape=jax.ShapeDtypeStruct((NPAD, D), jnp.float32),
    )(acc4, dis, b.reshape(1, D), res)


def _segmax_body(h_ref, b_ref, out_ref):
    i = pl.program_id(0)

    @pl.when(i == 0)
    def _():
        out_ref[...] = jnp.full((G, D), -jnp.inf, jnp.float32)

    bblk = b_ref[...]
    h = h_ref[...]
    g0 = bblk[0, 0]
    g1 = bblk[BN - 1, 0]

    def body(g, carry):
        m = jnp.where(bblk == g, h, -jnp.inf)
        row = jnp.max(m, axis=0, keepdims=True)
        out_ref[pl.ds(g, 1), :] = jnp.maximum(out_ref[pl.ds(g, 1), :], row)
        return carry

    lax.fori_loop(g0, g1 + 1, body, 0)


def _segmax(h, batch2d):
    return pl.pallas_call(
        _segmax_body,
        grid=(GRID,),
        in_specs=[
            pl.BlockSpec((BN, D), lambda i: (i, 0)),
            pl.BlockSpec((BN, 1), lambda i: (i, 0)),
        ],
        out_specs=pl.BlockSpec((G, D), lambda i: (0, 0)),
        out_shape=jax.ShapeDtypeStruct((G, D), jnp.float32),
    )(h, batch2d)


def kernel(x, edge_index, batch, W1, b1, W2, b2, W3, b3):
    x_pad = jnp.pad(x, ((0, NPAD - N), (0, 0)))
    pad_e = EPAD - E
    src3 = jnp.concatenate(
        [edge_index[0], jnp.full((pad_e,), N, jnp.int32)]).reshape(16, NB, BE)
    dst3 = jnp.concatenate(
        [edge_index[1], jnp.full((pad_e,), N, jnp.int32)]).reshape(16, NB, BE)
    batch2d = jnp.concatenate(
        [batch, jnp.full((NPAD - N,), batch[N - 1], batch.dtype)]
    ).reshape(NPAD, 1)

    ones_big = jnp.ones((NPAD, 128), jnp.float32)
    deg_big = _deg_kernel(dst3, ones_big)
    deg = deg_big[:, 0]
    dis = jnp.where(jnp.arange(NPAD) < N, lax.rsqrt(deg), 0.0)
    dis = dis.reshape(NPAD, 1).astype(jnp.float32)

    zeros_res = jnp.zeros((NPAD, D), jnp.float32)

    hs4 = _mm_scaled(x_pad, dis, W1)
    acc4 = _scatter_kernel(hs4, src3, dst3)
    z1 = _epilogue(acc4, dis, b1, zeros_res)

    hs4 = _mm_scaled(z1, dis, W2)
    acc4 = _scatter_kernel(hs4, src3, dst3)
    z2 = _epilogue(acc4, dis, b2, z1)

    hs4 = _mm_scaled(z2, dis, W3)
    acc4 = _scatter_kernel(hs4, src3, dst3)
    out_n = _epilogue(acc4, dis, b3, z2)

    g_feat = _segmax(out_n, batch2d)
    return out_n[:N], g_feat


# speedup vs baseline: 6.2321x; 1.2010x over previous
import functools

import jax
import jax.numpy as jnp
from jax import lax
from jax.experimental import pallas as pl
from jax.experimental.pallas import tpu as pltpu
from jax.experimental.pallas import tpu_sc as plsc

N = 10000
NPAD = 10008
D = 512
G = 64
E = 160000
EPAD = 163840
NB = 80
BE = 128
NBH = NB // 2
BN = 1112
GRID = NPAD // BN
NCHUNK = 4

def _make_mesh():
    return plsc.VectorSubcoreMesh(core_axis_name="c", subcore_axis_name="s")


def _deg_body(dst3, ones_big, out, didx, ones_v, shared, sem):
    del sem
    cid = lax.axis_index("c")
    sid = lax.axis_index("s")
    pltpu.sync_copy(dst3.at[sid], didx)
    pltpu.sync_copy(ones_big.at[pl.ds(0, BE)], ones_v)

    @pl.when(sid == 0)
    def _():
        pltpu.sync_copy(ones_big, shared)

    plsc.subcore_barrier()

    def body(t, carry):
        pltpu.sync_copy(ones_v, shared.at[didx.at[t]], add=True)
        return carry

    lax.fori_loop(0, NB, body, 0)
    plsc.subcore_barrier()

    @pl.when((sid == 0) & (cid == 0))
    def _():
        pltpu.sync_copy(shared, out)


def _deg_kernel(dst3, ones_big):
    fn = pl.kernel(
        _deg_body,
        mesh=_make_mesh(),
        out_type=jax.ShapeDtypeStruct((NPAD, 128), jnp.float32),
        scratch_types=[
            pltpu.VMEM((NB, BE), jnp.int32),
            pltpu.VMEM((BE, 128), jnp.float32),
            pltpu.VMEM_SHARED((NPAD, 128), jnp.float32),
            pltpu.SemaphoreType.DMA,
        ],
    )
    return fn(dst3, ones_big)


def _scatter_body(hs4, src3, dst3, out, sidx, didx, rows_a, rows_b, shared,
                  sem_a, sem_b):
    cid = lax.axis_index("c")
    sid = lax.axis_index("s")
    for jj in range(NCHUNK // 2):
        chunk = cid * (NCHUNK // 2) + jj

        @pl.when(sid == 0)
        def _():
            pltpu.sync_copy(hs4.at[chunk], shared)

        plsc.subcore_barrier()

        for half in range(NB // NBH):
            pltpu.sync_copy(src3.at[sid].at[pl.ds(half * NBH, NBH)], sidx)
            pltpu.sync_copy(dst3.at[sid].at[pl.ds(half * NBH, NBH)], didx)

            pltpu.async_copy(hs4.at[chunk].at[sidx.at[0]], rows_a, sem_a)

            def body(k, carry):
                ta = 2 * k
                tb = 2 * k + 1
                pltpu.async_copy(hs4.at[chunk].at[sidx.at[tb]], rows_b, sem_b)
                pltpu.make_async_copy(
                    hs4.at[chunk].at[sidx.at[ta]], rows_a, sem_a).wait()
                pltpu.sync_copy(rows_a, shared.at[didx.at[ta]], add=True)

                @pl.when(ta + 2 < NBH)
                def _():
                    pltpu.async_copy(
                        hs4.at[chunk].at[sidx.at[ta + 2]], rows_a, sem_a)

                pltpu.make_async_copy(
                    hs4.at[chunk].at[sidx.at[tb]], rows_b, sem_b).wait()
                pltpu.sync_copy(rows_b, shared.at[didx.at[tb]], add=True)
                return carry

            lax.fori_loop(0, NBH // 2, body, 0)
        plsc.subcore_barrier()

        @pl.when(sid == 0)
        def _():
            pltpu.sync_copy(shared, out.at[chunk])

        plsc.subcore_barrier()


def _scatter_kernel(hs4, src3, dst3):
    fn = pl.kernel(
        _scatter_body,
        mesh=_make_mesh(),
        out_type=jax.ShapeDtypeStruct((NCHUNK, NPAD, 128), jnp.float32),
        scratch_types=[
            pltpu.VMEM((NBH, BE), jnp.int32),
            pltpu.VMEM((NBH, BE), jnp.int32),
            pltpu.VMEM((BE, 128), jnp.float32),
            pltpu.VMEM((BE, 128), jnp.float32),
            pltpu.VMEM_SHARED((NPAD, 128), jnp.float32),
            pltpu.SemaphoreType.DMA,
            pltpu.SemaphoreType.DMA,
        ],
    )
    return fn(hs4, src3, dst3)


def _mm_body(x_ref, dis_ref, w_ref, out_ref):
    xs = x_ref[...] * dis_ref[...]
    h = jnp.dot(xs, w_ref[...], preferred_element_type=jnp.float32)
    for j in range(NCHUNK):
        out_ref[j] = h[:, j * 128:(j + 1) * 128]


def _mm_scaled(x, dis, w):
    return pl.pallas_call(
        _mm_body,
        grid=(GRID,),
        in_specs=[
            pl.BlockSpec((BN, D), lambda i: (i, 0)),
            pl.BlockSpec((BN, 1), lambda i: (i, 0)),
            pl.BlockSpec((D, D), lambda i: (0, 0)),
        ],
        out_specs=pl.BlockSpec((NCHUNK, BN, 128), lambda i: (0, i, 0)),
        out_shape=jax.ShapeDtypeStruct((NCHUNK, NPAD, 128), jnp.float32),
    )(x, dis, w)


def _ew_body(acc_ref, dis_ref, b_ref, res_ref, out_ref):
    h = jnp.concatenate([acc_ref[j] for j in range(NCHUNK)], axis=-1)
    out_ref[...] = res_ref[...] + jax.nn.relu(h * dis_ref[...] + b_ref[...])


def _epilogue(acc4, dis, b, res):
    return pl.pallas_call(
        _ew_body,
        grid=(GRID,),
        in_specs=[
            pl.BlockSpec((NCHUNK, BN, 128), lambda i: (0, i, 0)),
            pl.BlockSpec((BN, 1), lambda i: (i, 0)),
            pl.BlockSpec((1, D), lambda i: (0, 0)),
            pl.BlockSpec((BN, D), lambda i: (i, 0)),
        ],
        out_specs=pl.BlockSpec((BN, D), lambda i: (i, 0)),
        out_shape=jax.ShapeDtypeStruct((NPAD, D), jnp.float32),
    )(acc4, dis, b.reshape(1, D), res)


def _segmax_body(h_ref, b_ref, out_ref):
    i = pl.program_id(0)

    @pl.when(i == 0)
    def _():
        out_ref[...] = jnp.full((G, D), -jnp.inf, jnp.float32)

    bblk = b_ref[...]
    h = h_ref[...]
    g0 = bblk[0, 0]
    g1 = bblk[BN - 1, 0]

    def body(g, carry):
        m = jnp.where(bblk == g, h, -jnp.inf)
        row = jnp.max(m, axis=0, keepdims=True)
        out_ref[pl.ds(g, 1), :] = jnp.maximum(out_ref[pl.ds(g, 1), :], row)
        return carry

    lax.fori_loop(g0, g1 + 1, body, 0)


def _segmax(h, batch2d):
    return pl.pallas_call(
        _segmax_body,
        grid=(GRID,),
        in_specs=[
            pl.BlockSpec((BN, D), lambda i: (i, 0)),
            pl.BlockSpec((BN, 1), lambda i: (i, 0)),
        ],
        out_specs=pl.BlockSpec((G, D), lambda i: (0, 0)),
        out_shape=jax.ShapeDtypeStruct((G, D), jnp.float32),
    )(h, batch2d)


def kernel(x, edge_index, batch, W1, b1, W2, b2, W3, b3):
    x_pad = jnp.pad(x, ((0, NPAD - N), (0, 0)))
    pad_e = EPAD - E
    src3 = jnp.concatenate(
        [edge_index[0], jnp.full((pad_e,), N, jnp.int32)]).reshape(16, NB, BE)
    dst3 = jnp.concatenate(
        [edge_index[1], jnp.full((pad_e,), N, jnp.int32)]).reshape(16, NB, BE)
    batch2d = jnp.concatenate(
        [batch, jnp.full((NPAD - N,), batch[N - 1], batch.dtype)]
    ).reshape(NPAD, 1)

    ones_big = jnp.ones((NPAD, 128), jnp.float32)
    deg_big = _deg_kernel(dst3, ones_big)
    deg = deg_big[:, 0]
    dis = jnp.where(jnp.arange(NPAD) < N, lax.rsqrt(deg), 0.0)
    dis = dis.reshape(NPAD, 1).astype(jnp.float32)

    zeros_res = jnp.zeros((NPAD, D), jnp.float32)

    hs4 = _mm_scaled(x_pad, dis, W1)
    acc4 = _scatter_kernel(hs4, src3, dst3)
    z1 = _epilogue(acc4, dis, b1, zeros_res)

    hs4 = _mm_scaled(z1, dis, W2)
    acc4 = _scatter_kernel(hs4, src3, dst3)
    z2 = _epilogue(acc4, dis, b2, z1)

    hs4 = _mm_scaled(z2, dis, W3)
    acc4 = _scatter_kernel(hs4, src3, dst3)
    out_n = _epilogue(acc4, dis, b3, z2)

    g_feat = _segmax(out_n, batch2d)
    return out_n[:N], g_feat
